# baseline (speedup 1.0000x reference)
import jax
import jax.numpy as jnp
from jax import lax
from jax.experimental import pallas as pl
from jax.experimental.pallas import tpu as pltpu
from jax.experimental.pallas import tpu_sc as plsc

B = 16384
K = 128
NC = 2
NS = 16
NW = NC * NS
BPW = B // NW
CHUNK = 128
NCHUNK = BPW // CHUNK
KB = K // 16


def _afm_body(student_hbm, item_hbm, lo_hbm, q_hbm, alphas_hbm,
              betas_hbm, gammas_hbm, out_hbm,
              idx_s_v, idx_i_v, alpha_v, out_v, beta_v, gamma_v, tbuf,
              q_v0, lo_v0, q_v1, lo_v1, sem_a, sem_q0, sem_l0,
              sem_q1, sem_l1):
  wid = lax.axis_index("s") * NC + lax.axis_index("c")
  base = wid * BPW

  pltpu.sync_copy(student_hbm.at[pl.ds(base, BPW)], idx_s_v)
  cp_a = pltpu.async_copy(alphas_hbm.at[idx_s_v], alpha_v, sem_a)
  pltpu.sync_copy(item_hbm.at[pl.ds(base, BPW)], idx_i_v)
  pltpu.sync_copy(betas_hbm, beta_v)
  pltpu.sync_copy(gammas_hbm, gamma_v)

  bblks = [beta_v[pl.ds(kb * 16, 16)] for kb in range(KB)]
  gblks = [gamma_v[pl.ds(kb * 16, 16)] for kb in range(KB)]
  lane16 = lax.iota(jnp.int32, 16) * 16
  cols = []
  for sb in range(KB // 2):
    ce = sb * 32 + 2 * lax.iota(jnp.int32, 16)
    cols.append((ce, ce + 1))

  bufs = [(q_v0, lo_v0, sem_q0, sem_l0), (q_v1, lo_v1, sem_q1, sem_l1)]

  def issue(c):
    qb, lb, sq, sl = bufs[c % 2]
    row0 = c * CHUNK
    hq = pltpu.async_copy(
        q_hbm.at[idx_i_v.at[pl.ds(row0, CHUNK)]], qb, sq)
    lo_off = pl.multiple_of((base + row0) * (K // 2), 8)
    hl = pltpu.async_copy(
        lo_hbm.at[pl.ds(lo_off, CHUNK * K // 2)], lb, sl)
    return hq, hl

  pending = {0: issue(0)}

  for c in range(NCHUNK):
    if c + 1 < NCHUNK:
      pending[c + 1] = issue(c + 1)
    hq, hl = pending.pop(c)
    hq.wait()
    hl.wait()
    qb, lb, _, _ = bufs[c % 2]
    row0 = c * CHUNK

    def grp_body(g, _, qb=qb, lb=lb, row0=row0):
      rbase = g * 16
      for r in range(16):
        acc0 = acc1 = None
        rsplat = jnp.full((16,), rbase + r, jnp.int32)
        for sb in range(KB // 2):
          l16 = lb[pl.ds(((rbase + r) * K + sb * 32) // 2, 16)]
          l32 = plsc.bitcast(l16, jnp.bfloat16)
          l0, l1 = plsc.unpack(l32, format=plsc.PackFormat.INTERLEAVED)
          q0 = plsc.load_gather(qb, [rsplat, cols[sb][0]])
          q1 = plsc.load_gather(qb, [rsplat, cols[sb][1]])
          t0 = q0 * (bblks[2 * sb] + l0 * gblks[2 * sb])
          t1 = q1 * (bblks[2 * sb + 1] + l1 * gblks[2 * sb + 1])
          acc0 = t0 if acc0 is None else acc0 + t0
          acc1 = t1 if acc1 is None else acc1 + t1
        plsc.store_scatter(tbuf, [lane16 + r], acc0 + acc1)
      parts = [tbuf[pl.ds(l * 16, 16)] for l in range(16)]
      while len(parts) > 1:
        parts = [parts[i] + parts[i + 1] for i in range(0, len(parts), 2)]
      out_v[pl.ds(row0 + rbase, 16)] = parts[0]
      return 0

    lax.fori_loop(0, CHUNK // 16, grp_body, 0)

  cp_a.wait()

  def sig_body(i, _):
    s = pl.ds(i * 16, 16)
    logits = alpha_v[s] + out_v[s]
    out_v[s] = 1.0 / (1.0 + jnp.exp(-logits))
    return 0

  lax.fori_loop(0, BPW // 16, sig_body, 0, unroll=4)

  pltpu.sync_copy(out_v, out_hbm.at[pl.ds(base, BPW)])


@jax.jit
def _afm(student, item, lo_bf16, q_matrix, alphas, betas, gammas):
  mesh = plsc.VectorSubcoreMesh(core_axis_name="c", subcore_axis_name="s")
  kern = pl.kernel(
      _afm_body,
      out_type=jax.ShapeDtypeStruct((B,), jnp.float32),
      mesh=mesh,
      compiler_params=pltpu.CompilerParams(needs_layout_passes=False),
      scratch_types=[
          pltpu.VMEM((BPW,), jnp.int32),
          pltpu.VMEM((BPW,), jnp.int32),
          pltpu.VMEM((BPW,), jnp.float32),
          pltpu.VMEM((BPW,), jnp.float32),
          pltpu.VMEM((K,), jnp.float32),
          pltpu.VMEM((K,), jnp.float32),
          pltpu.VMEM((256,), jnp.float32),
          pltpu.VMEM((CHUNK, K), jnp.float32),
          pltpu.VMEM((CHUNK * K // 2,), jnp.int32),
          pltpu.VMEM((CHUNK, K), jnp.float32),
          pltpu.VMEM((CHUNK * K // 2,), jnp.int32),
          pltpu.SemaphoreType.DMA,
          pltpu.SemaphoreType.DMA,
          pltpu.SemaphoreType.DMA,
          pltpu.SemaphoreType.DMA,
          pltpu.SemaphoreType.DMA,
      ],
  )
  return kern(student, item, lo_bf16, q_matrix, alphas, betas, gammas)


def kernel(student, item, learning_opportunities, q_matrix,
           max_learning_opportunities, alphas, betas, gammas):
  del max_learning_opportunities
  lo_pairs = jax.lax.bitcast_convert_type(
      learning_opportunities.astype(jnp.bfloat16).reshape(B * K // 2, 2),
      jnp.int32)
  beta_p = betas.reshape(K // 32, 16, 2).transpose(0, 2, 1).reshape(K)
  gamma_p = gammas.reshape(K // 32, 16, 2).transpose(0, 2, 1).reshape(K)
  return _afm(student.astype(jnp.int32), item.astype(jnp.int32),
              lo_pairs, q_matrix, alphas, beta_p, gamma_p)


# speedup vs baseline: 16.3259x; 16.3259x over previous
import jax
import jax.numpy as jnp
from jax import lax
from jax.experimental import pallas as pl
from jax.experimental.pallas import tpu as pltpu
from jax.experimental.pallas import tpu_sc as plsc

B = 16384
K = 128
NC = 2
NS = 16
NW = NC * NS
BPW = B // NW
CHUNK = 128
NCHUNK = BPW // CHUNK
KB = K // 16


def _afm_body(student_hbm, item_hbm, lo_hbm, q_hbm, alphas_hbm,
              betas_hbm, gammas_hbm, out_hbm,
              idx_s_v, idx_i_v, alpha_v, out_v, beta_v, gamma_v, tbuf,
              tbuf2, q_v0, lo_v0, q_v1, lo_v1, sem_a, sem_q0, sem_l0,
              sem_q1, sem_l1):
  wid = lax.axis_index("s") * NC + lax.axis_index("c")
  base = wid * BPW

  pltpu.sync_copy(student_hbm.at[pl.ds(base, BPW)], idx_s_v)
  cp_a = pltpu.async_copy(alphas_hbm.at[idx_s_v], alpha_v, sem_a)
  pltpu.sync_copy(item_hbm.at[pl.ds(base, BPW)], idx_i_v)
  pltpu.sync_copy(betas_hbm, beta_v)
  pltpu.sync_copy(gammas_hbm, gamma_v)

  bblks = [beta_v[pl.ds(kb * 16, 16)] for kb in range(KB)]
  gblks = [gamma_v[pl.ds(kb * 16, 16)] for kb in range(KB)]
  lane16 = lax.iota(jnp.int32, 16) * 16

  bufs = [(q_v0, lo_v0, sem_q0, sem_l0), (q_v1, lo_v1, sem_q1, sem_l1)]

  def issue(c):
    qb, lb, sq, sl = bufs[c % 2]
    row0 = c * CHUNK
    hq = pltpu.async_copy(
        q_hbm.at[idx_i_v.at[pl.ds(row0, CHUNK)]], qb, sq)
    hl = pltpu.async_copy(
        lo_hbm.at[pl.ds(base + row0, CHUNK)], lb, sl)
    return hq, hl

  pending = {0: issue(0)}

  for c in range(NCHUNK):
    if c + 1 < NCHUNK:
      pending[c + 1] = issue(c + 1)
    hq, hl = pending.pop(c)
    hq.wait()
    hl.wait()
    qb, lb, _, _ = bufs[c % 2]
    row0 = c * CHUNK

    def pair_body(p, _, qb=qb, lb=lb, row0=row0):
      for half, tb in ((0, tbuf), (1, tbuf2)):
        rbase = p * 32 + half * 16
        for r in range(16):
          acc0 = acc1 = None
          for sb in range(KB // 2):
            ks0 = pl.ds(sb * 32, 16)
            ks1 = pl.ds(sb * 32 + 16, 16)
            q0 = qb[rbase + r, ks0]
            q1 = qb[rbase + r, ks1]
            l0 = lb[rbase + r, ks0].astype(jnp.float32)
            l1 = lb[rbase + r, ks1].astype(jnp.float32)
            t0 = q0 * (bblks[2 * sb] + l0 * gblks[2 * sb])
            t1 = q1 * (bblks[2 * sb + 1] + l1 * gblks[2 * sb + 1])
            acc0 = t0 if acc0 is None else acc0 + t0
            acc1 = t1 if acc1 is None else acc1 + t1
          plsc.store_scatter(tb, [lane16 + r], acc0 + acc1)
        parts = [tb[pl.ds(l * 16, 16)] for l in range(16)]
        while len(parts) > 1:
          parts = [parts[i] + parts[i + 1] for i in range(0, len(parts), 2)]
        out_v[pl.ds(row0 + rbase, 16)] = parts[0]
      return 0

    lax.fori_loop(0, CHUNK // 32, pair_body, 0)

  cp_a.wait()

  def sig_body(i, _):
    s = pl.ds(i * 16, 16)
    logits = alpha_v[s] + out_v[s]
    out_v[s] = 1.0 / (1.0 + jnp.exp(-logits))
    return 0

  lax.fori_loop(0, BPW // 16, sig_body, 0, unroll=4)

  pltpu.sync_copy(out_v, out_hbm.at[pl.ds(base, BPW)])


@jax.jit
def _afm(student, item, lo_i32, q_matrix, alphas, betas, gammas):
  mesh = plsc.VectorSubcoreMesh(core_axis_name="c", subcore_axis_name="s")
  kern = pl.kernel(
      _afm_body,
      out_type=jax.ShapeDtypeStruct((B,), jnp.float32),
      mesh=mesh,
      compiler_params=pltpu.CompilerParams(needs_layout_passes=False),
      scratch_types=[
          pltpu.VMEM((BPW,), jnp.int32),
          pltpu.VMEM((BPW,), jnp.int32),
          pltpu.VMEM((BPW,), jnp.float32),
          pltpu.VMEM((BPW,), jnp.float32),
          pltpu.VMEM((K,), jnp.float32),
          pltpu.VMEM((K,), jnp.float32),
          pltpu.VMEM((256,), jnp.float32),
          pltpu.VMEM((256,), jnp.float32),
          pltpu.VMEM((CHUNK, K), jnp.float32),
          pltpu.VMEM((CHUNK, K), jnp.int32),
          pltpu.VMEM((CHUNK, K), jnp.float32),
          pltpu.VMEM((CHUNK, K), jnp.int32),
          pltpu.SemaphoreType.DMA,
          pltpu.SemaphoreType.DMA,
          pltpu.SemaphoreType.DMA,
          pltpu.SemaphoreType.DMA,
          pltpu.SemaphoreType.DMA,
      ],
  )
  return kern(student, item, lo_i32, q_matrix, alphas, betas, gammas)


def kernel(student, item, learning_opportunities, q_matrix,
           max_learning_opportunities, alphas, betas, gammas):
  del max_learning_opportunities
  return _afm(student.astype(jnp.int32), item.astype(jnp.int32),
              learning_opportunities, q_matrix, alphas, betas, gammas)


# speedup vs baseline: 18.9059x; 1.1580x over previous
import jax
import jax.numpy as jnp
from jax import lax
from jax.experimental import pallas as pl
from jax.experimental.pallas import tpu as pltpu
from jax.experimental.pallas import tpu_sc as plsc

B = 16384
K = 128
NC = 2
NS = 16
NW = NC * NS
BPW = B // NW
CHUNK = 128
NCHUNK = BPW // CHUNK
KB = K // 16


def _afm_body(student_hbm, item_hbm, lo_hbm, q_hbm, alphas_hbm,
              betas_hbm, gammas_hbm, out_hbm,
              idx_s_v, idx_i_v, alpha_v, out_v, beta_v, gamma_v, tbuf,
              q_v0, lo_v0, q_v1, lo_v1, sem_a, sem_i, sem_st, sem_q0,
              sem_l0, sem_q1, sem_l1):
  wid = lax.axis_index("s") * NC + lax.axis_index("c")
  base = wid * BPW

  pltpu.sync_copy(item_hbm.at[pl.ds(base, BPW)], idx_i_v)

  bufs = [(q_v0, lo_v0, sem_q0, sem_l0), (q_v1, lo_v1, sem_q1, sem_l1)]

  def issue(c):
    qb, lb, sq, sl = bufs[c % 2]
    row0 = c * CHUNK
    hq = pltpu.async_copy(
        q_hbm.at[idx_i_v.at[pl.ds(row0, CHUNK)]], qb, sq)
    hl = pltpu.async_copy(
        lo_hbm.at[pl.ds(base + row0, CHUNK)], lb, sl)
    return hq, hl

  pending = {0: issue(0)}
  pltpu.sync_copy(student_hbm.at[pl.ds(base, BPW)], idx_s_v)
  pltpu.sync_copy(betas_hbm, beta_v)
  pltpu.sync_copy(gammas_hbm, gamma_v)
  cp_a = pltpu.async_copy(alphas_hbm.at[idx_s_v], alpha_v, sem_a)

  bblks = [beta_v[pl.ds(kb * 16, 16)] for kb in range(KB)]
  gblks = [gamma_v[pl.ds(kb * 16, 16)] for kb in range(KB)]
  lane16 = lax.iota(jnp.int32, 16) * 16

  for c in range(NCHUNK):
    if c + 1 < NCHUNK:
      pending[c + 1] = issue(c + 1)
    hq, hl = pending.pop(c)
    hq.wait()
    hl.wait()
    qb, lb, _, _ = bufs[c % 2]
    row0 = c * CHUNK

    def pair_body(p, _, qb=qb, lb=lb, row0=row0):
      for half in (0, 1):
        rbase = p * 32 + half * 16
        tb0 = p * 512 + half * 256
        for r in range(0, 16, 2):
          accs = [[None, None], [None, None]]
          for sb in range(KB // 2):
            ks0 = pl.ds(sb * 32, 16)
            ks1 = pl.ds(sb * 32 + 16, 16)
            for j in (0, 1):
              q0 = qb[rbase + r + j, ks0]
              q1 = qb[rbase + r + j, ks1]
              l0 = lb[rbase + r + j, ks0].astype(jnp.float32)
              l1 = lb[rbase + r + j, ks1].astype(jnp.float32)
              t0 = q0 * (bblks[2 * sb] + l0 * gblks[2 * sb])
              t1 = q1 * (bblks[2 * sb + 1] + l1 * gblks[2 * sb + 1])
              a = accs[j]
              a[0] = t0 if a[0] is None else a[0] + t0
              a[1] = t1 if a[1] is None else a[1] + t1
          for j in (0, 1):
            plsc.store_scatter(tbuf, [tb0 + lane16 + r + j],
                               accs[j][0] + accs[j][1])
        parts = [tbuf[pl.ds(tb0 + l * 16, 16)] for l in range(16)]
        while len(parts) > 1:
          parts = [parts[i] + parts[i + 1] for i in range(0, len(parts), 2)]
        out_v[pl.ds(row0 + rbase, 16)] = parts[0]
      return 0

    lax.fori_loop(0, CHUNK // 32, pair_body, 0)

  cp_a.wait()

  def sig_body(i, _):
    s = pl.ds(i * 16, 16)
    logits = alpha_v[s] + out_v[s]
    out_v[s] = 1.0 / (1.0 + jnp.exp(-logits))
    return 0

  lax.fori_loop(0, BPW // 16, sig_body, 0, unroll=4)

  pltpu.sync_copy(out_v, out_hbm.at[pl.ds(base, BPW)])


@jax.jit
def _afm(student, item, lo_i32, q_matrix, alphas, betas, gammas):
  mesh = plsc.VectorSubcoreMesh(core_axis_name="c", subcore_axis_name="s")
  kern = pl.kernel(
      _afm_body,
      out_type=jax.ShapeDtypeStruct((B,), jnp.float32),
      mesh=mesh,
      compiler_params=pltpu.CompilerParams(needs_layout_passes=False),
      scratch_types=[
          pltpu.VMEM((BPW,), jnp.int32),
          pltpu.VMEM((BPW,), jnp.int32),
          pltpu.VMEM((BPW,), jnp.float32),
          pltpu.VMEM((BPW,), jnp.float32),
          pltpu.VMEM((K,), jnp.float32),
          pltpu.VMEM((K,), jnp.float32),
          pltpu.VMEM((CHUNK * 16,), jnp.float32),
          pltpu.VMEM((CHUNK, K), jnp.float32),
          pltpu.VMEM((CHUNK, K), jnp.int32),
          pltpu.VMEM((CHUNK, K), jnp.float32),
          pltpu.VMEM((CHUNK, K), jnp.int32),
          pltpu.SemaphoreType.DMA,
          pltpu.SemaphoreType.DMA,
          pltpu.SemaphoreType.DMA,
          pltpu.SemaphoreType.DMA,
          pltpu.SemaphoreType.DMA,
          pltpu.SemaphoreType.DMA,
          pltpu.SemaphoreType.DMA,
      ],
  )
  return kern(student, item, lo_i32, q_matrix, alphas, betas, gammas)


def kernel(student, item, learning_opportunities, q_matrix,
           max_learning_opportunities, alphas, betas, gammas):
  del max_learning_opportunities
  return _afm(student.astype(jnp.int32), item.astype(jnp.int32),
              learning_opportunities, q_matrix, alphas, betas, gammas)


# speedup vs baseline: 19.5736x; 1.0353x over previous
import jax
import jax.numpy as jnp
from jax import lax
from jax.experimental import pallas as pl
from jax.experimental.pallas import tpu as pltpu
from jax.experimental.pallas import tpu_sc as plsc

B = 16384
K = 128
NC = 2
NS = 16
NW = NC * NS
BPW = B // NW
CHUNK = 128
NCHUNK = BPW // CHUNK
KB = K // 16


def _afm_body(student_hbm, item_hbm, lo_hbm, q_hbm, alphas_hbm,
              betas_hbm, gammas_hbm, out_hbm,
              idx_s_v, idx_i_v, alpha_v, out_v, beta_v, gamma_v, tbuf,
              q_v0, lo_v0, q_v1, lo_v1, sem_a, sem_i, sem_st, sem_q0,
              sem_l0, sem_q1, sem_l1):
  wid = lax.axis_index("s") * NC + lax.axis_index("c")
  base = wid * BPW

  pltpu.sync_copy(item_hbm.at[pl.ds(base, BPW)], idx_i_v)

  bufs = [(q_v0, lo_v0, sem_q0, sem_l0), (q_v1, lo_v1, sem_q1, sem_l1)]

  def issue(c):
    qb, lb, sq, sl = bufs[c % 2]
    row0 = c * CHUNK
    hq = pltpu.async_copy(
        q_hbm.at[idx_i_v.at[pl.ds(row0, CHUNK)]], qb, sq)
    hl = pltpu.async_copy(
        lo_hbm.at[pl.ds(base + row0, CHUNK)], lb, sl)
    return hq, hl

  pending = {0: issue(0)}
  pltpu.sync_copy(student_hbm.at[pl.ds(base, BPW)], idx_s_v)
  pltpu.sync_copy(betas_hbm, beta_v)
  pltpu.sync_copy(gammas_hbm, gamma_v)
  cp_a = pltpu.async_copy(alphas_hbm.at[idx_s_v], alpha_v, sem_a)

  bblks = [beta_v[pl.ds(kb * 16, 16)] for kb in range(KB)]
  gblks = [gamma_v[pl.ds(kb * 16, 16)] for kb in range(KB)]
  lane16 = lax.iota(jnp.int32, 16) * 16

  for c in range(NCHUNK):
    if c + 1 < NCHUNK:
      pending[c + 1] = issue(c + 1)
    hq, hl = pending.pop(c)
    hq.wait()
    hl.wait()
    qb, lb, _, _ = bufs[c % 2]
    row0 = c * CHUNK

    def pair_body(p, _, qb=qb, lb=lb, row0=row0):
      for half in (0, 1):
        rbase = p * 32 + half * 16
        tb0 = p * 512 + half * 256
        for r in range(0, 16, 4):
          accs = [[None, None] for _ in range(4)]
          for sb in range(KB // 2):
            ks0 = pl.ds(sb * 32, 16)
            ks1 = pl.ds(sb * 32 + 16, 16)
            for j in (0, 1, 2, 3):
              q0 = qb[rbase + r + j, ks0]
              q1 = qb[rbase + r + j, ks1]
              l0 = lb[rbase + r + j, ks0].astype(jnp.float32)
              l1 = lb[rbase + r + j, ks1].astype(jnp.float32)
              t0 = q0 * (bblks[2 * sb] + l0 * gblks[2 * sb])
              t1 = q1 * (bblks[2 * sb + 1] + l1 * gblks[2 * sb + 1])
              a = accs[j]
              a[0] = t0 if a[0] is None else a[0] + t0
              a[1] = t1 if a[1] is None else a[1] + t1
          for j in (0, 1, 2, 3):
            plsc.store_scatter(tbuf, [tb0 + lane16 + r + j],
                               accs[j][0] + accs[j][1])
        parts = [tbuf[pl.ds(tb0 + l * 16, 16)] for l in range(16)]
        while len(parts) > 1:
          parts = [parts[i] + parts[i + 1] for i in range(0, len(parts), 2)]
        out_v[pl.ds(row0 + rbase, 16)] = parts[0]
      return 0

    lax.fori_loop(0, CHUNK // 32, pair_body, 0)

  cp_a.wait()

  def sig_body(i, _):
    s = pl.ds(i * 16, 16)
    logits = alpha_v[s] + out_v[s]
    out_v[s] = 1.0 / (1.0 + jnp.exp(-logits))
    return 0

  lax.fori_loop(0, BPW // 16, sig_body, 0, unroll=4)

  pltpu.sync_copy(out_v, out_hbm.at[pl.ds(base, BPW)])


@jax.jit
def _afm(student, item, lo_i32, q_matrix, alphas, betas, gammas):
  mesh = plsc.VectorSubcoreMesh(core_axis_name="c", subcore_axis_name="s")
  kern = pl.kernel(
      _afm_body,
      out_type=jax.ShapeDtypeStruct((B,), jnp.float32),
      mesh=mesh,
      compiler_params=pltpu.CompilerParams(needs_layout_passes=False),
      scratch_types=[
          pltpu.VMEM((BPW,), jnp.int32),
          pltpu.VMEM((BPW,), jnp.int32),
          pltpu.VMEM((BPW,), jnp.float32),
          pltpu.VMEM((BPW,), jnp.float32),
          pltpu.VMEM((K,), jnp.float32),
          pltpu.VMEM((K,), jnp.float32),
          pltpu.VMEM((CHUNK * 16,), jnp.float32),
          pltpu.VMEM((CHUNK, K), jnp.float32),
          pltpu.VMEM((CHUNK, K), jnp.int32),
          pltpu.VMEM((CHUNK, K), jnp.float32),
          pltpu.VMEM((CHUNK, K), jnp.int32),
          pltpu.SemaphoreType.DMA,
          pltpu.SemaphoreType.DMA,
          pltpu.SemaphoreType.DMA,
          pltpu.SemaphoreType.DMA,
          pltpu.SemaphoreType.DMA,
          pltpu.SemaphoreType.DMA,
          pltpu.SemaphoreType.DMA,
      ],
  )
  return kern(student, item, lo_i32, q_matrix, alphas, betas, gammas)


def kernel(student, item, learning_opportunities, q_matrix,
           max_learning_opportunities, alphas, betas, gammas):
  del max_learning_opportunities
  return _afm(student.astype(jnp.int32), item.astype(jnp.int32),
              learning_opportunities, q_matrix, alphas, betas, gammas)
